# baseline (speedup 1.0000x reference)
import functools

import jax
import jax.numpy as jnp
from jax import lax
from jax.experimental import pallas as pl
from jax.experimental.pallas import tpu as pltpu
from jax.experimental.pallas import tpu_sc as plsc

GRID_CELLS = 900
NUM_COLORS = 11
EMBED = 128

_NC, _NS = 2, 16
NW = _NC * _NS
TABLE_PAD = ((GRID_CELLS * NUM_COLORS + 8 * _NS - 1) // (8 * _NS)) * (8 * _NS)


def _table_body(ct_ref, pe_ref, w1_ref, w2_ref, b_ref, out_ref):
    cp = lax.dot_general(ct_ref[...], w1_ref[...], (((1,), (1,)), ((), ())),
                         preferred_element_type=jnp.float32)
    pp = lax.dot_general(pe_ref[...], w2_ref[...], (((1,), (1,)), ((), ())),
                         preferred_element_type=jnp.float32)
    out_ref[...] = pp[:, None, :] + cp[None, :, :] + b_ref[...][None, None, :]


def _build_table(color_table, pos_embedding, w1, w2, b):
    return pl.pallas_call(
        _table_body,
        out_shape=jax.ShapeDtypeStruct((GRID_CELLS, NUM_COLORS, EMBED),
                                       jnp.float32),
    )(color_table, pos_embedding, w1, w2, b)


def _make_gather(total_rows: int, chunk: int, max_idx: int = 128):
    rows_per_w = total_rows // NW
    n_chunks = rows_per_w // chunk
    assert rows_per_w * NW == total_rows and n_chunks * chunk == rows_per_w
    assert chunk % 16 == 0
    n_pairs = n_chunks // 2
    tail = n_chunks % 2
    slices = []
    off = 0
    while off < chunk:
        sl = min(max_idx, chunk - off)
        slices.append((off, sl))
        off += sl

    mesh = plsc.VectorSubcoreMesh(core_axis_name="c", subcore_axis_name="s")

    @functools.partial(
        pl.kernel,
        mesh=mesh,
        out_type=jax.ShapeDtypeStruct((total_rows, EMBED), jnp.float32),
        scratch_types=[
            pltpu.VMEM((chunk,), jnp.int32),
            pltpu.VMEM((chunk,), jnp.int32),
            pltpu.VMEM((chunk,), jnp.int32),
            pltpu.VMEM((chunk,), jnp.int32),
            pltpu.VMEM((chunk, EMBED), jnp.float32),
            pltpu.VMEM((chunk, EMBED), jnp.float32),
            pltpu.SemaphoreType.DMA,
            pltpu.SemaphoreType.DMA,
            pltpu.SemaphoreType.DMA,
            pltpu.SemaphoreType.DMA,
            pltpu.VMEM_SHARED((TABLE_PAD, EMBED),
                              jnp.float32),
        ],
    )
    def gather(grids_hbm, table_hbm, out_hbm,
               grid_a, grid_b, idx_a, idx_b, rows_a, rows_b,
               sga, sgb, swa, swb, table_sp):
        sid = lax.axis_index("s")
        wid = sid * _NC + lax.axis_index("c")
        base_w = wid * rows_per_w
        lane = lax.iota(jnp.int32, 16)

        slab = TABLE_PAD // _NS
        pltpu.sync_copy(table_hbm.at[pl.ds(sid * slab, slab)],
                        table_sp.at[pl.ds(sid * slab, slab)])
        plsc.subcore_barrier()

        def stage_idx(coff, grid_v, idx_v):
            pltpu.sync_copy(grids_hbm.at[pl.ds(base_w + coff, chunk)],
                            grid_v)
            for i in range(chunk // 16):
                pos = base_w + coff + (i * 16) + lane
                j = lax.rem(pos, GRID_CELLS)
                idx_v[pl.ds(i * 16, 16)] = (
                    j * NUM_COLORS + grid_v[pl.ds(i * 16, 16)])

        def fire_gathers(idx_v, rows_v, sem):
            return [
                pltpu.async_copy(
                    table_sp.at[idx_v.at[pl.ds(o, sl)]],
                    rows_v.at[pl.ds(o, sl)], sem)
                for (o, sl) in slices
            ]

        def drain_write(rows_v, sem):
            pltpu.make_async_copy(
                rows_v, out_hbm.at[pl.ds(0, chunk)], sem).wait()

        def one_pair(t, carry):
            c0 = (2 * t) * chunk
            c1 = c0 + chunk
            stage_idx(c0, grid_a, idx_a)

            @pl.when(t > 0)
            def _():
                drain_write(rows_a, swa)

            ha = fire_gathers(idx_a, rows_a, sga)
            stage_idx(c1, grid_b, idx_b)

            @pl.when(t > 0)
            def _():
                drain_write(rows_b, swb)

            hb = fire_gathers(idx_b, rows_b, sgb)
            for h in ha:
                h.wait()
            pltpu.async_copy(rows_a, out_hbm.at[pl.ds(base_w + c0, chunk)],
                             swa)
            for h in hb:
                h.wait()
            pltpu.async_copy(rows_b, out_hbm.at[pl.ds(base_w + c1, chunk)],
                             swb)
            return carry

        lax.fori_loop(0, n_pairs, one_pair, 0)
        if tail:
            ct = (n_chunks - 1) * chunk
            stage_idx(ct, grid_a, idx_a)
            drain_write(rows_a, swa)
            ha = fire_gathers(idx_a, rows_a, sga)
            for h in ha:
                h.wait()
            pltpu.async_copy(rows_a, out_hbm.at[pl.ds(base_w + ct, chunk)],
                             swa)
        drain_write(rows_a, swa)
        drain_write(rows_b, swb)

    return gather


def kernel(grids, color_table, pos_embedding, W, b):
    batch, num_grids, h, w = grids.shape
    total_rows = batch * num_grids * h * w
    w1 = W[:, : EMBED // 2]
    w2 = W[:, EMBED // 2:]
    table = _build_table(color_table, pos_embedding, w1, w2, b)
    table2d = table.reshape(GRID_CELLS * NUM_COLORS, EMBED)
    table2d = jnp.pad(table2d,
                      ((0, TABLE_PAD - GRID_CELLS * NUM_COLORS), (0, 0)))
    g1d = grids.reshape(-1).astype(jnp.int32)
    out = _make_gather(total_rows, 192, max_idx=192)(g1d, table2d)
    return out.reshape(batch, num_grids * h * w, EMBED)


# speedup vs baseline: 1.0049x; 1.0049x over previous
import functools

import jax
import jax.numpy as jnp
from jax import lax
from jax.experimental import pallas as pl
from jax.experimental.pallas import tpu as pltpu
from jax.experimental.pallas import tpu_sc as plsc

GRID_CELLS = 900
NUM_COLORS = 11
EMBED = 128

_NC, _NS = 2, 16
NW = _NC * _NS
TABLE_PAD = GRID_CELLS * NUM_COLORS


def _table_body(ct_ref, pe_ref, w1_ref, w2_ref, b_ref, out_ref):
    cp = lax.dot_general(ct_ref[...], w1_ref[...], (((1,), (1,)), ((), ())),
                         preferred_element_type=jnp.float32)
    pp = lax.dot_general(pe_ref[...], w2_ref[...], (((1,), (1,)), ((), ())),
                         preferred_element_type=jnp.float32)
    out_ref[...] = pp[:, None, :] + cp[None, :, :] + b_ref[...][None, None, :]


def _build_table(color_table, pos_embedding, w1, w2, b):
    return pl.pallas_call(
        _table_body,
        out_shape=jax.ShapeDtypeStruct((GRID_CELLS, NUM_COLORS, EMBED),
                                       jnp.float32),
    )(color_table, pos_embedding, w1, w2, b)


def _make_gather(total_rows: int, chunk: int, max_idx: int = 128):
    rows_per_w = total_rows // NW
    n_chunks = rows_per_w // chunk
    assert rows_per_w * NW == total_rows and n_chunks * chunk == rows_per_w
    assert chunk % 16 == 0
    n_pairs = n_chunks // 2
    tail = n_chunks % 2
    slices = []
    off = 0
    while off < chunk:
        sl = min(max_idx, chunk - off)
        slices.append((off, sl))
        off += sl

    mesh = plsc.VectorSubcoreMesh(core_axis_name="c", subcore_axis_name="s")

    @functools.partial(
        pl.kernel,
        mesh=mesh,
        out_type=jax.ShapeDtypeStruct((total_rows, EMBED), jnp.float32),
        scratch_types=[
            pltpu.VMEM((chunk,), jnp.int32),
            pltpu.VMEM((chunk,), jnp.int32),
            pltpu.VMEM((chunk,), jnp.int32),
            pltpu.VMEM((chunk,), jnp.int32),
            pltpu.VMEM((chunk, EMBED), jnp.float32),
            pltpu.VMEM((chunk, EMBED), jnp.float32),
            pltpu.SemaphoreType.DMA,
            pltpu.SemaphoreType.DMA,
            pltpu.SemaphoreType.DMA,
            pltpu.SemaphoreType.DMA,
            pltpu.VMEM_SHARED((TABLE_PAD, EMBED),
                              jnp.float32),
        ],
    )
    def gather(grids_hbm, table_hbm, out_hbm,
               grid_a, grid_b, idx_a, idx_b, rows_a, rows_b,
               sga, sgb, swa, swb, table_sp):
        sid = lax.axis_index("s")
        wid = sid * _NC + lax.axis_index("c")
        lane = lax.iota(jnp.int32, 16)

        @pl.when(sid == 0)
        def _():
            pltpu.sync_copy(table_hbm, table_sp)

        plsc.subcore_barrier()

        def stage_idx(abs_base, grid_v, idx_v):
            pltpu.sync_copy(grids_hbm.at[pl.ds(abs_base, chunk)], grid_v)
            for i in range(chunk // 16):
                pos = abs_base + (i * 16) + lane
                j = lax.rem(pos, GRID_CELLS)
                idx_v[pl.ds(i * 16, 16)] = (
                    j * NUM_COLORS + grid_v[pl.ds(i * 16, 16)])

        def fire_gathers(idx_v, rows_v, sem):
            return [
                pltpu.async_copy(
                    table_sp.at[idx_v.at[pl.ds(o, sl)]],
                    rows_v.at[pl.ds(o, sl)], sem)
                for (o, sl) in slices
            ]

        def drain_write(rows_v, sem):
            pltpu.make_async_copy(
                rows_v, out_hbm.at[pl.ds(0, chunk)], sem).wait()

        def chunk_base(ci):
            return (ci * NW + wid) * chunk

        def one_pair(t, carry):
            b0 = chunk_base(2 * t)
            b1 = chunk_base(2 * t + 1)
            stage_idx(b0, grid_a, idx_a)

            @pl.when(t > 0)
            def _():
                drain_write(rows_a, swa)

            ha = fire_gathers(idx_a, rows_a, sga)
            stage_idx(b1, grid_b, idx_b)

            @pl.when(t > 0)
            def _():
                drain_write(rows_b, swb)

            hb = fire_gathers(idx_b, rows_b, sgb)
            for h in ha:
                h.wait()
            pltpu.async_copy(rows_a, out_hbm.at[pl.ds(b0, chunk)], swa)
            for h in hb:
                h.wait()
            pltpu.async_copy(rows_b, out_hbm.at[pl.ds(b1, chunk)], swb)
            return carry

        lax.fori_loop(0, n_pairs, one_pair, 0)
        if tail:
            bt = chunk_base(n_chunks - 1)
            stage_idx(bt, grid_a, idx_a)
            drain_write(rows_a, swa)
            ha = fire_gathers(idx_a, rows_a, sga)
            for h in ha:
                h.wait()
            pltpu.async_copy(rows_a, out_hbm.at[pl.ds(bt, chunk)], swa)
        drain_write(rows_a, swa)
        drain_write(rows_b, swb)

    return gather


def kernel(grids, color_table, pos_embedding, W, b):
    batch, num_grids, h, w = grids.shape
    total_rows = batch * num_grids * h * w
    w1 = W[:, : EMBED // 2]
    w2 = W[:, EMBED // 2:]
    table = _build_table(color_table, pos_embedding, w1, w2, b)
    table2d = table.reshape(GRID_CELLS * NUM_COLORS, EMBED)
    g1d = grids.reshape(-1).astype(jnp.int32)
    out = _make_gather(total_rows, 192, max_idx=192)(g1d, table2d)
    return out.reshape(batch, num_grids * h * w, EMBED)
